# baseline (speedup 1.0000x reference)
import functools

import jax
import jax.numpy as jnp
from jax import lax
from jax.experimental import pallas as pl
from jax.experimental.pallas import tpu as pltpu
from jax.experimental.pallas import tpu_sc as plsc

F = 26
D = 16
BUCKET = 1020
BPAD = 1024
FIELD_DIM = 40000
B = 16384
NW = 32
BPW = B // NW
NBLK = BPW // 16
LIN_CHUNK = 128
NCHUNK = F * BPW // LIN_CHUNK


def _sc_body(x_hbm, q_hbm, r_hbm, qs_hbm, rs_hbm, bias_hbm, lin_hbm, out_hbm,
             x_v, qraw, rraw, qs_v, rs_v, bias_v, qtab, rtab,
             qbuf, rbuf, gidx, linbuf, fm_v, out_v, sem):
    cid = lax.axis_index("c")
    sid = lax.axis_index("s")
    wid = sid * 2 + cid
    base = wid * BPW

    pltpu.sync_copy(x_hbm.at[pl.ds(base * F, BPW * F)], x_v)
    pltpu.sync_copy(q_hbm, qraw.at[pl.ds(0, BUCKET * D)])
    pltpu.sync_copy(r_hbm, rraw.at[pl.ds(0, BUCKET * D)])
    pltpu.sync_copy(qs_hbm, qs_v)
    pltpu.sync_copy(rs_hbm, rs_v)
    pltpu.sync_copy(bias_hbm, bias_v)

    lane = lax.iota(jnp.int32, 16)
    tq = 1.0 / (1.0 + jnp.exp(-qs_v[:]))
    tr = 1.0 / (1.0 + jnp.exp(-rs_v[:]))

    def _thr(v, t):
        return jnp.sign(v) * jnp.maximum(jnp.abs(v) - t, 0.0)

    @plsc.parallel_loop(0, BPAD // 16)
    def tprep(c):
        rowbase = (lane + c * 16) * D
        for k in range(D // 2):
            ql = _thr(plsc.load_gather(qraw, [rowbase + 2 * k]), tq)
            qh = _thr(plsc.load_gather(qraw, [rowbase + 2 * k + 1]), tq)
            rl = _thr(plsc.load_gather(rraw, [rowbase + 2 * k]), tr)
            rh = _thr(plsc.load_gather(rraw, [rowbase + 2 * k + 1]), tr)
            qw = plsc.bitcast(
                plsc.pack(ql, qh, format=plsc.PackFormat.INTERLEAVED),
                jnp.int32)
            rw = plsc.bitcast(
                plsc.pack(rl, rh, format=plsc.PackFormat.INTERLEAVED),
                jnp.int32)
            qtab[pl.ds(k * BPAD + c * 16, 16)] = qw
            rtab[pl.ds(k * BPAD + c * 16, 16)] = rw

    @plsc.parallel_loop(0, NBLK)
    def prep(blk):
        s0 = blk * 16
        bvec = (lane + s0) * F
        for f in range(F):
            g = plsc.load_gather(x_v, [bvec + f]) + f * FIELD_DIM
            q = lax.div(g, BUCKET)
            r = g - q * BUCKET
            qbuf[f, pl.ds(s0, 16)] = q
            rbuf[f, pl.ds(s0, 16)] = r
            p = (f * NBLK + blk) * 16
            gidx[p // LIN_CHUNK, pl.ds(p % LIN_CHUNK, 16)] = g

    def fire(j, _):
        pltpu.make_async_copy(lin_hbm.at[gidx.at[j]], linbuf.at[j], sem).start()
        return 0

    lax.fori_loop(0, NCHUNK, fire, 0)

    @plsc.parallel_loop(0, NBLK)
    def fm(blk):
        s0 = blk * 16
        acc_s = [jnp.zeros((16,), jnp.float32) for _ in range(D)]
        sqtot = jnp.zeros((16,), jnp.float32)
        himask = jnp.full((16,), -0x10000, jnp.int32)
        for f in range(F):
            qi = qbuf[f, pl.ds(s0, 16)]
            ri = rbuf[f, pl.ds(s0, 16)]
            for k in range(D // 2):
                wq = plsc.load_gather(qtab.at[pl.ds(k * BPAD, BPAD)], [qi])
                wr = plsc.load_gather(rtab.at[pl.ds(k * BPAD, BPAD)], [ri])
                qa = plsc.bitcast(lax.shift_left(wq, 16), jnp.float32)
                qb = plsc.bitcast(wq & himask, jnp.float32)
                ra = plsc.bitcast(lax.shift_left(wr, 16), jnp.float32)
                rb = plsc.bitcast(wr & himask, jnp.float32)
                e0 = qa * ra
                e1 = qb * rb
                acc_s[2 * k] = acc_s[2 * k] + e0
                acc_s[2 * k + 1] = acc_s[2 * k + 1] + e1
                sqtot = sqtot + e0 * e0
                sqtot = sqtot + e1 * e1
        tot = acc_s[0] * acc_s[0]
        for d in range(1, D):
            tot = tot + acc_s[d] * acc_s[d]
        fm_v[pl.ds(s0, 16)] = 0.5 * (tot - sqtot)

    def drain(j, _):
        pltpu.make_async_copy(lin_hbm.at[gidx.at[j]], linbuf.at[j], sem).wait()
        return 0

    lax.fori_loop(0, NCHUNK, drain, 0)

    @plsc.parallel_loop(0, NBLK)
    def lin(blk):
        s0 = blk * 16
        acc = bias_v[:] + fm_v[pl.ds(s0, 16)]
        for f in range(F):
            row = (f * BPW) // LIN_CHUNK
            acc = acc + linbuf[row + blk // 8, pl.ds((blk % 8) * 16, 16)]
        out_v[pl.ds(s0, 16)] = acc

    pltpu.sync_copy(out_v, out_hbm.at[pl.ds(base, BPW)])


@functools.partial(
    pl.kernel,
    out_type=jax.ShapeDtypeStruct((B,), jnp.float32),
    mesh=plsc.VectorSubcoreMesh(core_axis_name="c", subcore_axis_name="s"),
    compiler_params=pltpu.CompilerParams(needs_layout_passes=False),
    scratch_types=[
        pltpu.VMEM((F * BPW,), jnp.int32),
        pltpu.VMEM((BPAD * D,), jnp.float32),
        pltpu.VMEM((BPAD * D,), jnp.float32),
        pltpu.VMEM((16,), jnp.float32),
        pltpu.VMEM((16,), jnp.float32),
        pltpu.VMEM((16,), jnp.float32),
        pltpu.VMEM((D // 2 * BPAD,), jnp.int32),
        pltpu.VMEM((D // 2 * BPAD,), jnp.int32),
        pltpu.VMEM((F, BPW), jnp.int32),
        pltpu.VMEM((F, BPW), jnp.int32),
        pltpu.VMEM((NCHUNK, LIN_CHUNK), jnp.int32),
        pltpu.VMEM((NCHUNK, LIN_CHUNK), jnp.float32),
        pltpu.VMEM((BPW,), jnp.float32),
        pltpu.VMEM((BPW,), jnp.float32),
        pltpu.SemaphoreType.DMA,
    ],
)
def _sc_kernel(*refs):
    _sc_body(*refs)


def kernel(x, Q_v, R_v, Q_s, R_s, lin_w, lin_b, offsets):
    del offsets
    qs_v = jnp.full((16,), Q_s, jnp.float32)
    rs_v = jnp.full((16,), R_s, jnp.float32)
    bias_v = jnp.full((16,), lin_b[0], jnp.float32)
    return _sc_kernel(x.reshape(-1), Q_v.reshape(-1), R_v.reshape(-1),
                      qs_v, rs_v, bias_v, lin_w.reshape(-1))


# speedup vs baseline: 1.0736x; 1.0736x over previous
import functools

import jax
import jax.numpy as jnp
from jax import lax
from jax.experimental import pallas as pl
from jax.experimental.pallas import tpu as pltpu
from jax.experimental.pallas import tpu_sc as plsc

F = 26
D = 16
BUCKET = 1020
BPAD = 1024
FIELD_DIM = 40000
B = 16384
NW = 32
BPW = B // NW
NBLK = BPW // 16
LIN_CHUNK = 128
NCHUNK = F * BPW // LIN_CHUNK


def _pack_rows(t):
    lo = lax.bitcast_convert_type(t[0:8, :].astype(jnp.bfloat16),
                                  jnp.uint16).astype(jnp.uint32)
    hi = lax.bitcast_convert_type(t[8:16, :].astype(jnp.bfloat16),
                                  jnp.uint16).astype(jnp.uint32)
    return lax.bitcast_convert_type(lo | (hi << 16), jnp.int32)


def _thresh_body(q_ref, r_ref, s_ref, oq_ref, or_ref):
    tq = jax.nn.sigmoid(s_ref[0, 0])
    tr = jax.nn.sigmoid(s_ref[0, 1])
    q = q_ref[...]
    r = r_ref[...]
    oq_ref[...] = _pack_rows(jnp.sign(q) * jnp.maximum(jnp.abs(q) - tq, 0.0))
    or_ref[...] = _pack_rows(jnp.sign(r) * jnp.maximum(jnp.abs(r) - tr, 0.0))


def _threshold_tables(qT, rT, Q_s, R_s):
    scal = jnp.stack([Q_s, R_s]).reshape(1, 2).astype(jnp.float32)
    return pl.pallas_call(
        _thresh_body,
        out_shape=(
            jax.ShapeDtypeStruct((D // 2, BPAD), jnp.int32),
            jax.ShapeDtypeStruct((D // 2, BPAD), jnp.int32),
        ),
        in_specs=[
            pl.BlockSpec(memory_space=pltpu.VMEM),
            pl.BlockSpec(memory_space=pltpu.VMEM),
            pl.BlockSpec(memory_space=pltpu.SMEM),
        ],
        out_specs=(
            pl.BlockSpec(memory_space=pltpu.VMEM),
            pl.BlockSpec(memory_space=pltpu.VMEM),
        ),
    )(qT, rT, scal)


def _sc_body(x_hbm, qt_hbm, rt_hbm, bias_hbm, lin_hbm, out_hbm,
             x_v, qtab, rtab, bias_v, qbuf, rbuf, gidx, linbuf, fm_v, out_v,
             sem):
    cid = lax.axis_index("c")
    sid = lax.axis_index("s")
    wid = sid * 2 + cid
    base = wid * BPW

    pltpu.sync_copy(x_hbm.at[pl.ds(base * F, BPW * F)], x_v)
    pltpu.sync_copy(qt_hbm, qtab)
    pltpu.sync_copy(rt_hbm, rtab)
    pltpu.sync_copy(bias_hbm, bias_v)

    lane = lax.iota(jnp.int32, 16)

    @plsc.parallel_loop(0, NBLK)
    def prep(blk):
        s0 = blk * 16
        bvec = (lane + s0) * F
        for f in range(F):
            g = plsc.load_gather(x_v, [bvec + f]) + f * FIELD_DIM
            q = lax.div(g, BUCKET)
            r = g - q * BUCKET
            qbuf[f, pl.ds(s0, 16)] = q
            rbuf[f, pl.ds(s0, 16)] = r
            p = (f * NBLK + blk) * 16
            gidx[p // LIN_CHUNK, pl.ds(p % LIN_CHUNK, 16)] = g

    def fire(j, _):
        pltpu.make_async_copy(lin_hbm.at[gidx.at[j]], linbuf.at[j], sem).start()
        return 0

    lax.fori_loop(0, NCHUNK, fire, 0)

    @plsc.parallel_loop(0, NBLK)
    def fm(blk):
        s0 = blk * 16
        acc_s = [jnp.zeros((16,), jnp.float32) for _ in range(D)]
        sqtot = jnp.zeros((16,), jnp.float32)
        himask = jnp.full((16,), -0x10000, jnp.int32)
        for f in range(F):
            qi = qbuf[f, pl.ds(s0, 16)]
            ri = rbuf[f, pl.ds(s0, 16)]
            for k in range(D // 2):
                wq = plsc.load_gather(qtab.at[pl.ds(k * BPAD, BPAD)], [qi])
                wr = plsc.load_gather(rtab.at[pl.ds(k * BPAD, BPAD)], [ri])
                pw = plsc.bitcast(plsc.bitcast(wq, jnp.bfloat16)
                                  * plsc.bitcast(wr, jnp.bfloat16), jnp.int32)
                e0 = plsc.bitcast(lax.shift_left(pw, 16), jnp.float32)
                e1 = plsc.bitcast(pw & himask, jnp.float32)
                acc_s[2 * k] = acc_s[2 * k] + e0
                acc_s[2 * k + 1] = acc_s[2 * k + 1] + e1
                sqtot = sqtot + e0 * e0
                sqtot = sqtot + e1 * e1
        tot = acc_s[0] * acc_s[0]
        for d in range(1, D):
            tot = tot + acc_s[d] * acc_s[d]
        fm_v[pl.ds(s0, 16)] = 0.5 * (tot - sqtot)

    def drain(j, _):
        pltpu.make_async_copy(lin_hbm.at[gidx.at[j]], linbuf.at[j], sem).wait()
        return 0

    lax.fori_loop(0, NCHUNK, drain, 0)

    @plsc.parallel_loop(0, NBLK)
    def lin(blk):
        s0 = blk * 16
        acc = bias_v[:] + fm_v[pl.ds(s0, 16)]
        for f in range(F):
            row = (f * BPW) // LIN_CHUNK
            acc = acc + linbuf[row + blk // 8, pl.ds((blk % 8) * 16, 16)]
        out_v[pl.ds(s0, 16)] = acc

    pltpu.sync_copy(out_v, out_hbm.at[pl.ds(base, BPW)])


@functools.partial(
    pl.kernel,
    out_type=jax.ShapeDtypeStruct((B,), jnp.float32),
    mesh=plsc.VectorSubcoreMesh(core_axis_name="c", subcore_axis_name="s"),
    compiler_params=pltpu.CompilerParams(needs_layout_passes=False),
    scratch_types=[
        pltpu.VMEM((F * BPW,), jnp.int32),
        pltpu.VMEM((D // 2 * BPAD,), jnp.int32),
        pltpu.VMEM((D // 2 * BPAD,), jnp.int32),
        pltpu.VMEM((16,), jnp.float32),
        pltpu.VMEM((F, BPW), jnp.int32),
        pltpu.VMEM((F, BPW), jnp.int32),
        pltpu.VMEM((NCHUNK, LIN_CHUNK), jnp.int32),
        pltpu.VMEM((NCHUNK, LIN_CHUNK), jnp.float32),
        pltpu.VMEM((BPW,), jnp.float32),
        pltpu.VMEM((BPW,), jnp.float32),
        pltpu.SemaphoreType.DMA,
    ],
)
def _sc_kernel(*refs):
    _sc_body(*refs)


def kernel(x, Q_v, R_v, Q_s, R_s, lin_w, lin_b, offsets):
    del offsets
    qT = jnp.pad(Q_v.T, ((0, 0), (0, BPAD - BUCKET)))
    rT = jnp.pad(R_v.T, ((0, 0), (0, BPAD - BUCKET)))
    qt, rt = _threshold_tables(qT, rT, Q_s, R_s)
    bias_v = jnp.full((16,), lin_b[0], jnp.float32)
    lin_flat = lin_w.reshape(-1)
    return _sc_kernel(x.reshape(-1), qt.reshape(-1), rt.reshape(-1),
                      bias_v, lin_flat)


# speedup vs baseline: 1.0930x; 1.0181x over previous
import functools

import jax
import jax.numpy as jnp
from jax import lax
from jax.experimental import pallas as pl
from jax.experimental.pallas import tpu as pltpu
from jax.experimental.pallas import tpu_sc as plsc

F = 26
D = 16
BUCKET = 1020
BPAD = 1024
FIELD_DIM = 40000
B = 16384
NW = 32
BPW = B // NW
NBLK = BPW // 16
LIN_CHUNK = 128
NCHUNK = F * BPW // LIN_CHUNK


def _pack_rows(t):
    lo = lax.bitcast_convert_type(t[0:8, :].astype(jnp.bfloat16),
                                  jnp.uint16).astype(jnp.uint32)
    hi = lax.bitcast_convert_type(t[8:16, :].astype(jnp.bfloat16),
                                  jnp.uint16).astype(jnp.uint32)
    return lax.bitcast_convert_type(lo | (hi << 16), jnp.int32)


def _thresh_body(q_ref, r_ref, s_ref, oq_ref, or_ref):
    tq = jax.nn.sigmoid(s_ref[0, 0])
    tr = jax.nn.sigmoid(s_ref[0, 1])
    q = q_ref[...]
    r = r_ref[...]
    oq_ref[...] = _pack_rows(jnp.sign(q) * jnp.maximum(jnp.abs(q) - tq, 0.0))
    or_ref[...] = _pack_rows(jnp.sign(r) * jnp.maximum(jnp.abs(r) - tr, 0.0))


def _threshold_tables(qT, rT, Q_s, R_s):
    scal = jnp.stack([Q_s, R_s]).reshape(1, 2).astype(jnp.float32)
    return pl.pallas_call(
        _thresh_body,
        out_shape=(
            jax.ShapeDtypeStruct((D // 2, BPAD), jnp.int32),
            jax.ShapeDtypeStruct((D // 2, BPAD), jnp.int32),
        ),
        in_specs=[
            pl.BlockSpec(memory_space=pltpu.VMEM),
            pl.BlockSpec(memory_space=pltpu.VMEM),
            pl.BlockSpec(memory_space=pltpu.SMEM),
        ],
        out_specs=(
            pl.BlockSpec(memory_space=pltpu.VMEM),
            pl.BlockSpec(memory_space=pltpu.VMEM),
        ),
    )(qT, rT, scal)


def _sc_body(x_hbm, qt_hbm, rt_hbm, bias_hbm, lin_hbm, out_hbm,
             x_v, qtab, rtab, bias_v, qbuf, rbuf, gidx, linbuf, fm_v, out_v,
             sem, sem_tab):
    cid = lax.axis_index("c")
    sid = lax.axis_index("s")
    wid = sid * 2 + cid
    base = wid * BPW

    cq = pltpu.make_async_copy(qt_hbm, qtab, sem_tab)
    cr = pltpu.make_async_copy(rt_hbm, rtab, sem_tab)
    cb = pltpu.make_async_copy(bias_hbm, bias_v, sem_tab)
    cq.start()
    cr.start()
    cb.start()
    pltpu.sync_copy(x_hbm.at[pl.ds(base * F, BPW * F)], x_v)

    lane = lax.iota(jnp.int32, 16)

    @plsc.parallel_loop(0, NBLK)
    def prep(blk):
        s0 = blk * 16
        bvec = (lane + s0) * F
        for f in range(F):
            g = plsc.load_gather(x_v, [bvec + f]) + f * FIELD_DIM
            q = lax.div(g, BUCKET)
            r = g - q * BUCKET
            qbuf[f, pl.ds(s0, 16)] = q
            rbuf[f, pl.ds(s0, 16)] = r
            p = (f * NBLK + blk) * 16
            gidx[p // LIN_CHUNK, pl.ds(p % LIN_CHUNK, 16)] = g

    def fire(j, _):
        pltpu.make_async_copy(lin_hbm.at[gidx.at[j]],
                              linbuf.at[pl.ds(j * LIN_CHUNK, LIN_CHUNK)],
                              sem).start()
        return 0

    lax.fori_loop(0, NCHUNK, fire, 0)

    cq.wait()
    cr.wait()
    cb.wait()

    @plsc.parallel_loop(0, NBLK)
    def fm(blk):
        s0 = blk * 16
        acc_s = [jnp.zeros((16,), jnp.float32) for _ in range(D)]
        sqtot = jnp.zeros((16,), jnp.float32)
        himask = jnp.full((16,), -0x10000, jnp.int32)
        for f in range(F):
            qi = qbuf[f, pl.ds(s0, 16)]
            ri = rbuf[f, pl.ds(s0, 16)]
            for k in range(D // 2):
                wq = plsc.load_gather(qtab.at[pl.ds(k * BPAD, BPAD)], [qi])
                wr = plsc.load_gather(rtab.at[pl.ds(k * BPAD, BPAD)], [ri])
                pw = plsc.bitcast(plsc.bitcast(wq, jnp.bfloat16)
                                  * plsc.bitcast(wr, jnp.bfloat16), jnp.int32)
                e0 = plsc.bitcast(lax.shift_left(pw, 16), jnp.float32)
                e1 = plsc.bitcast(pw & himask, jnp.float32)
                acc_s[2 * k] = acc_s[2 * k] + e0
                acc_s[2 * k + 1] = acc_s[2 * k + 1] + e1
                sqtot = sqtot + e0 * e0
                sqtot = sqtot + e1 * e1
        tot = acc_s[0] * acc_s[0]
        for d in range(1, D):
            tot = tot + acc_s[d] * acc_s[d]
        fm_v[pl.ds(s0, 16)] = 0.5 * (tot - sqtot)

    pltpu.make_async_copy(lin_hbm.at[pl.ds(0, NCHUNK * LIN_CHUNK)],
                          linbuf, sem).wait()

    @plsc.parallel_loop(0, NBLK)
    def lin(blk):
        s0 = blk * 16
        acc = bias_v[:] + fm_v[pl.ds(s0, 16)]
        for f in range(F):
            acc = acc + linbuf[pl.ds(f * BPW + s0, 16)]
        out_v[pl.ds(s0, 16)] = acc

    pltpu.sync_copy(out_v, out_hbm.at[pl.ds(base, BPW)])


@functools.partial(
    pl.kernel,
    out_type=jax.ShapeDtypeStruct((B,), jnp.float32),
    mesh=plsc.VectorSubcoreMesh(core_axis_name="c", subcore_axis_name="s"),
    compiler_params=pltpu.CompilerParams(needs_layout_passes=False),
    scratch_types=[
        pltpu.VMEM((F * BPW,), jnp.int32),
        pltpu.VMEM((D // 2 * BPAD,), jnp.int32),
        pltpu.VMEM((D // 2 * BPAD,), jnp.int32),
        pltpu.VMEM((16,), jnp.float32),
        pltpu.VMEM((F, BPW), jnp.int32),
        pltpu.VMEM((F, BPW), jnp.int32),
        pltpu.VMEM((NCHUNK, LIN_CHUNK), jnp.int32),
        pltpu.VMEM((NCHUNK * LIN_CHUNK,), jnp.float32),
        pltpu.VMEM((BPW,), jnp.float32),
        pltpu.VMEM((BPW,), jnp.float32),
        pltpu.SemaphoreType.DMA,
        pltpu.SemaphoreType.DMA,
    ],
)
def _sc_kernel(*refs):
    _sc_body(*refs)


def kernel(x, Q_v, R_v, Q_s, R_s, lin_w, lin_b, offsets):
    del offsets
    qT = jnp.pad(Q_v.T, ((0, 0), (0, BPAD - BUCKET)))
    rT = jnp.pad(R_v.T, ((0, 0), (0, BPAD - BUCKET)))
    qt, rt = _threshold_tables(qT, rT, Q_s, R_s)
    bias_v = jnp.full((16,), lin_b[0], jnp.float32)
    lin_flat = lin_w.reshape(-1)
    return _sc_kernel(x.reshape(-1), qt.reshape(-1), rt.reshape(-1),
                      bias_v, lin_flat)


# speedup vs baseline: 1.4306x; 1.3089x over previous
import functools

import jax
import jax.numpy as jnp
from jax import lax
from jax.experimental import pallas as pl
from jax.experimental.pallas import tpu as pltpu
from jax.experimental.pallas import tpu_sc as plsc

F = 26
D = 16
BUCKET = 1020
BPAD = 1024
FIELD_DIM = 40000
B = 16384
NW = 32
BPW = B // NW
NBLK = BPW // 16
LIN_CHUNK = 128
NCHUNK = F * BPW // LIN_CHUNK


def _pack_rows(t):
    lo = lax.bitcast_convert_type(t[0:8, :].astype(jnp.bfloat16),
                                  jnp.uint16).astype(jnp.uint32)
    hi = lax.bitcast_convert_type(t[8:16, :].astype(jnp.bfloat16),
                                  jnp.uint16).astype(jnp.uint32)
    return lax.bitcast_convert_type(lo | (hi << 16), jnp.int32)


def _thresh_body(q_ref, r_ref, s_ref, oq_ref, or_ref):
    tq = jax.nn.sigmoid(s_ref[0, 0])
    tr = jax.nn.sigmoid(s_ref[0, 1])
    q = q_ref[...]
    r = r_ref[...]
    oq_ref[...] = _pack_rows(jnp.sign(q) * jnp.maximum(jnp.abs(q) - tq, 0.0))
    or_ref[...] = _pack_rows(jnp.sign(r) * jnp.maximum(jnp.abs(r) - tr, 0.0))


def _threshold_tables(qT, rT, Q_s, R_s):
    scal = jnp.stack([Q_s, R_s]).reshape(1, 2).astype(jnp.float32)
    return pl.pallas_call(
        _thresh_body,
        out_shape=(
            jax.ShapeDtypeStruct((D // 2, BPAD), jnp.int32),
            jax.ShapeDtypeStruct((D // 2, BPAD), jnp.int32),
        ),
        in_specs=[
            pl.BlockSpec(memory_space=pltpu.VMEM),
            pl.BlockSpec(memory_space=pltpu.VMEM),
            pl.BlockSpec(memory_space=pltpu.SMEM),
        ],
        out_specs=(
            pl.BlockSpec(memory_space=pltpu.VMEM),
            pl.BlockSpec(memory_space=pltpu.VMEM),
        ),
    )(qT, rT, scal)


def _sc_body(x_hbm, qt_hbm, rt_hbm, bias_hbm, lin_hbm, out_hbm,
             x_v, qtab, rtab, bias_v, qbuf, rbuf, gidx, linbuf, fm_v, out_v,
             sem, sem_tab):
    cid = lax.axis_index("c")
    sid = lax.axis_index("s")
    wid = sid * 2 + cid
    base = wid * BPW

    cq = pltpu.make_async_copy(qt_hbm, qtab, sem_tab)
    cr = pltpu.make_async_copy(rt_hbm, rtab, sem_tab)
    cb = pltpu.make_async_copy(bias_hbm, bias_v, sem_tab)
    cq.start()
    cr.start()
    cb.start()
    pltpu.sync_copy(x_hbm.at[pl.ds(base * F, BPW * F)], x_v)

    lane = lax.iota(jnp.int32, 16)

    @plsc.parallel_loop(0, NBLK)
    def prep(blk):
        s0 = blk * 16
        bvec = (lane + s0) * F
        for f in range(F):
            g = plsc.load_gather(x_v, [bvec + f]) + f * FIELD_DIM
            q = ((g.astype(jnp.float32) + 0.5)
                 * jnp.float32(1.0 / BUCKET)).astype(jnp.int32)
            r = g - q * BUCKET
            qbuf[f, pl.ds(s0, 16)] = q
            rbuf[f, pl.ds(s0, 16)] = r
            p = (f * NBLK + blk) * 16
            gidx[p // LIN_CHUNK, pl.ds(p % LIN_CHUNK, 16)] = g

    def fire(j, _):
        pltpu.make_async_copy(lin_hbm.at[gidx.at[j]],
                              linbuf.at[pl.ds(j * LIN_CHUNK, LIN_CHUNK)],
                              sem).start()
        return 0

    lax.fori_loop(0, NCHUNK, fire, 0)

    cq.wait()
    cr.wait()
    cb.wait()

    @plsc.parallel_loop(0, NBLK)
    def fm(blk):
        s0 = blk * 16
        acc_s = [jnp.zeros((16,), jnp.float32) for _ in range(D)]
        sqtot = jnp.zeros((16,), jnp.float32)
        himask = jnp.full((16,), -0x10000, jnp.int32)
        for f in range(F):
            qi = qbuf[f, pl.ds(s0, 16)]
            ri = rbuf[f, pl.ds(s0, 16)]
            for k in range(D // 2):
                wq = plsc.load_gather(qtab.at[pl.ds(k * BPAD, BPAD)], [qi])
                wr = plsc.load_gather(rtab.at[pl.ds(k * BPAD, BPAD)], [ri])
                pw = plsc.bitcast(plsc.bitcast(wq, jnp.bfloat16)
                                  * plsc.bitcast(wr, jnp.bfloat16), jnp.int32)
                e0 = plsc.bitcast(lax.shift_left(pw, 16), jnp.float32)
                e1 = plsc.bitcast(pw & himask, jnp.float32)
                acc_s[2 * k] = acc_s[2 * k] + e0
                acc_s[2 * k + 1] = acc_s[2 * k + 1] + e1
                sqtot = sqtot + e0 * e0
                sqtot = sqtot + e1 * e1
        tot = acc_s[0] * acc_s[0]
        for d in range(1, D):
            tot = tot + acc_s[d] * acc_s[d]
        fm_v[pl.ds(s0, 16)] = 0.5 * (tot - sqtot)

    pltpu.make_async_copy(lin_hbm.at[pl.ds(0, NCHUNK * LIN_CHUNK)],
                          linbuf, sem).wait()

    @plsc.parallel_loop(0, NBLK)
    def lin(blk):
        s0 = blk * 16
        acc = bias_v[:] + fm_v[pl.ds(s0, 16)]
        for f in range(F):
            acc = acc + linbuf[pl.ds(f * BPW + s0, 16)]
        out_v[pl.ds(s0, 16)] = acc

    pltpu.sync_copy(out_v, out_hbm.at[pl.ds(base, BPW)])


@functools.partial(
    pl.kernel,
    out_type=jax.ShapeDtypeStruct((B,), jnp.float32),
    mesh=plsc.VectorSubcoreMesh(core_axis_name="c", subcore_axis_name="s"),
    compiler_params=pltpu.CompilerParams(needs_layout_passes=False),
    scratch_types=[
        pltpu.VMEM((F * BPW,), jnp.int32),
        pltpu.VMEM((D // 2 * BPAD,), jnp.int32),
        pltpu.VMEM((D // 2 * BPAD,), jnp.int32),
        pltpu.VMEM((16,), jnp.float32),
        pltpu.VMEM((F, BPW), jnp.int32),
        pltpu.VMEM((F, BPW), jnp.int32),
        pltpu.VMEM((NCHUNK, LIN_CHUNK), jnp.int32),
        pltpu.VMEM((NCHUNK * LIN_CHUNK,), jnp.float32),
        pltpu.VMEM((BPW,), jnp.float32),
        pltpu.VMEM((BPW,), jnp.float32),
        pltpu.SemaphoreType.DMA,
        pltpu.SemaphoreType.DMA,
    ],
)
def _sc_kernel(*refs):
    _sc_body(*refs)


def kernel(x, Q_v, R_v, Q_s, R_s, lin_w, lin_b, offsets):
    del offsets
    qT = jnp.pad(Q_v.T, ((0, 0), (0, BPAD - BUCKET)))
    rT = jnp.pad(R_v.T, ((0, 0), (0, BPAD - BUCKET)))
    qt, rt = _threshold_tables(qT, rT, Q_s, R_s)
    bias_v = jnp.full((16,), lin_b[0], jnp.float32)
    lin_flat = lin_w.reshape(-1)
    return _sc_kernel(x.reshape(-1), qt.reshape(-1), rt.reshape(-1),
                      bias_v, lin_flat)
